# baseline (speedup 1.0000x reference)
import functools

import jax
import jax.numpy as jnp
from jax import lax
from jax.experimental import pallas as pl
from jax.experimental.pallas import tpu as pltpu
from jax.experimental.pallas import tpu_sc as plsc

_NUM_CORES = 2
_NUM_SUBCORES = 16
_NUM_WORKERS = _NUM_CORES * _NUM_SUBCORES
_LANES = 16

_B = 64


def _odd(x):
    return x if x % 2 else x + 1


@functools.lru_cache(maxsize=None)
def _build_sc_lookup(n, words, tbytes, qdim, chunk, cent_rows):
    outd = qdim * chunk
    cstride = _odd(outd)
    twords = tbytes // 4
    per_w = n // _NUM_WORKERS
    n_chunks = per_w // _B
    assert per_w * _NUM_WORKERS == n and n_chunks * _B == per_w and n_chunks >= 6

    mesh = plsc.VectorSubcoreMesh(core_axis_name="c", subcore_axis_name="s")

    @functools.partial(
        pl.kernel,
        out_type=jax.ShapeDtypeStruct((n, outd), jnp.float32),
        mesh=mesh,
        scratch_types=[
            pltpu.VMEM((cent_rows * cstride,), jnp.float32),
            pltpu.VMEM((per_w,), jnp.int32),
            pltpu.VMEM((_B, twords), jnp.int32),
            pltpu.VMEM((_B, twords), jnp.int32),
            pltpu.VMEM((_B, outd), jnp.float32),
            pltpu.VMEM((_B, outd), jnp.float32),
            pltpu.SemaphoreType.DMA,
            pltpu.SemaphoreType.DMA,
            pltpu.SemaphoreType.DMA,
            pltpu.SemaphoreType.DMA,
            pltpu.SemaphoreType.DMA,
            pltpu.SemaphoreType.DMA,
        ],
        compiler_params=pltpu.CompilerParams(
            needs_layout_passes=False, use_tc_tiling_on_sc=False),
    )
    def sc_lookup(ids_hbm, table_hbm, codes_hbm, out_hbm,
                  codes_v, ids_v, rows0, rows1, out0, out1,
                  sem_c, sem_i, gsem0, gsem1, osem0, osem1):
        wid = lax.axis_index("s") * _NUM_CORES + lax.axis_index("c")
        base = wid * per_w

        rows = (rows0, rows1)
        outs = (out0, out1)
        gsems = (gsem0, gsem1)
        osems = (osem0, osem1)

        ccp = pltpu.async_copy(codes_hbm, codes_v, sem_c)
        pltpu.async_copy(ids_hbm.at[pl.ds(base, per_w)], ids_v, sem_i).wait()

        def start_gather(i, par):
            return pltpu.async_copy(
                table_hbm.at[ids_v.at[pl.ds(i * _B, _B)]],
                rows[par], gsems[par])

        def wait_gather(par):
            pltpu.make_async_copy(
                table_hbm.at[ids_v.at[pl.ds(0, _B)]],
                rows[par], gsems[par]).wait()

        def start_out(i, par):
            return pltpu.async_copy(
                outs[par], out_hbm.at[pl.ds(base + i * _B, _B)], osems[par])

        def wait_out(par):
            pltpu.make_async_copy(
                outs[par], out_hbm.at[pl.ds(0, _B)], osems[par]).wait()

        iot = lax.iota(jnp.int32, _LANES)
        n_groups = _B // _LANES
        rowvec = [iot + g * _LANES for g in range(n_groups)]
        rowvec_w = [rv * twords for rv in rowvec]

        def compute_chunk(par):
            rref = rows[par]
            oref = outs[par]

            def wbody(w, carry):
                w3 = w * (4 * chunk)
                colv = [jnp.broadcast_to(w3 + j, (_LANES,))
                        for j in range(4 * chunk)]
                wsplat = jnp.broadcast_to(w, (_LANES,))
                for g in range(n_groups):
                    packed = plsc.load_gather(rref, [rowvec[g], wsplat])
                    for sub in range(4):
                        b = packed if sub == 0 else (packed >> (8 * sub))
                        b = b & 0xFF
                        colbase = w3 + sub * chunk
                        coff = b * cstride + colbase
                        for c in range(chunk):
                            src = coff if c == 0 else coff + c
                            val = plsc.load_gather(codes_v, [src])
                            plsc.store_scatter(
                                oref, [rowvec[g], colv[sub * chunk + c]], val)
                return carry

            lax.fori_loop(0, words, wbody, jnp.int32(0))

        start_gather(0, 0)
        start_gather(1, 1)
        ccp.wait()

        for i in range(2):
            wait_gather(i)
            compute_chunk(i)
            start_out(i, i)
            start_gather(i + 2, i)

        def pbody(p, carry):
            for par in range(2):
                i = 2 * p + par
                wait_gather(par)
                wait_out(par)
                compute_chunk(par)
                start_out(i, par)
                start_gather(i + 2, par)
            return carry

        lax.fori_loop(1, n_chunks // 2 - 1, pbody, jnp.int32(0))

        for k in range(2):
            i = n_chunks - 2 + k
            par = i % 2
            wait_gather(par)
            wait_out(par)
            compute_chunk(par)
            start_out(i, par)

        wait_out(0)
        wait_out(1)

    return sc_lookup


def kernel(input, indexes, codes):
    shape = input.shape
    n = input.size
    vocab_rows, qdim = indexes.shape
    cent_rows, qdim2, chunk = codes.shape
    assert qdim == qdim2 and qdim % 4 == 0
    words = qdim // 4
    outd = qdim * chunk

    tbytes = (qdim + 63) // 64 * 64

    flat_ids = input.reshape(-1).astype(jnp.int32)
    table = jnp.pad(indexes, ((0, 0), (0, tbytes - qdim)))
    table = sum(table[:, k::4].astype(jnp.int32) << (8 * k)
                for k in range(4))
    codes_flat = jnp.pad(
        codes.reshape(cent_rows, outd),
        ((0, 0), (0, _odd(outd) - outd))).reshape(-1)

    fn = _build_sc_lookup(n, words, tbytes, qdim, chunk, cent_rows)
    out = fn(flat_ids, table, codes_flat)
    return out.reshape(shape + (outd,))


# speedup vs baseline: 7.7339x; 7.7339x over previous
import functools

import jax
import jax.numpy as jnp
from jax import lax
from jax.experimental import pallas as pl
from jax.experimental.pallas import tpu as pltpu
from jax.experimental.pallas import tpu_sc as plsc

_NUM_CORES = 2
_NUM_SUBCORES = 16
_NUM_WORKERS = _NUM_CORES * _NUM_SUBCORES
_LANES = 16

_B = 32
_PACK_BLK = 512


def _odd(x):
    return x if x % 2 else x + 1


@functools.lru_cache(maxsize=None)
def _build_packer(vocab_rows, qdim, rbytes):
    rwords = rbytes
    grid = (vocab_rows + _PACK_BLK - 1) // _PACK_BLK
    out_rows = grid * (_PACK_BLK // 4)

    def body(x_ref, o_ref):
        o_ref[...] = pltpu.bitcast(x_ref[...], jnp.int32)

    return pl.pallas_call(
        body,
        grid=(grid,),
        in_specs=[pl.BlockSpec((_PACK_BLK, rbytes), lambda i: (i, 0))],
        out_specs=pl.BlockSpec((_PACK_BLK // 4, rwords), lambda i: (i, 0)),
        out_shape=jax.ShapeDtypeStruct((out_rows, rwords), jnp.int32),
    )


@functools.lru_cache(maxsize=None)
def _build_sc_lookup(n, qdim, chunk, rwords, cent_rows):
    outd = qdim * chunk
    cstride = _odd(outd)
    per_w = n // _NUM_WORKERS
    n_chunks = per_w // _B
    assert per_w * _NUM_WORKERS == n and n_chunks * _B == per_w and n_chunks >= 6

    mesh = plsc.VectorSubcoreMesh(core_axis_name="c", subcore_axis_name="s")

    @functools.partial(
        pl.kernel,
        out_type=jax.ShapeDtypeStruct((n, outd), jnp.float32),
        mesh=mesh,
        scratch_types=[
            pltpu.VMEM((cent_rows * cstride,), jnp.float32),
            pltpu.VMEM((per_w,), jnp.int32),
            pltpu.VMEM((_B,), jnp.int32),
            pltpu.VMEM((_B,), jnp.int32),
            pltpu.VMEM((_B, rwords), jnp.int32),
            pltpu.VMEM((_B, rwords), jnp.int32),
            pltpu.VMEM((_B, outd), jnp.float32),
            pltpu.VMEM((_B, outd), jnp.float32),
            pltpu.SemaphoreType.DMA,
            pltpu.SemaphoreType.DMA,
            pltpu.SemaphoreType.DMA,
            pltpu.SemaphoreType.DMA,
            pltpu.SemaphoreType.DMA,
            pltpu.SemaphoreType.DMA,
        ],
        compiler_params=pltpu.CompilerParams(
            needs_layout_passes=False, use_tc_tiling_on_sc=False),
    )
    def sc_lookup(ids_hbm, table_hbm, codes_hbm, out_hbm,
                  codes_v, ids_v, recs0, recs1, rows0, rows1, out0, out1,
                  sem_c, sem_i, gsem0, gsem1, osem0, osem1):
        wid = lax.axis_index("s") * _NUM_CORES + lax.axis_index("c")
        base = wid * per_w

        recs = (recs0, recs1)
        rows = (rows0, rows1)
        outs = (out0, out1)
        gsems = (gsem0, gsem1)
        osems = (osem0, osem1)

        ccp = pltpu.async_copy(codes_hbm, codes_v, sem_c)
        pltpu.async_copy(ids_hbm.at[pl.ds(base, per_w)], ids_v, sem_i).wait()

        iot = lax.iota(jnp.int32, _LANES)
        n_groups = _B // _LANES
        rowvec = [iot + g * _LANES for g in range(n_groups)]

        def start_gather(i, par):
            for g in range(n_groups):
                ids_g = ids_v[pl.ds(i * _B + g * _LANES, _LANES)]
                recs[par][pl.ds(g * _LANES, _LANES)] = ids_g >> 2
            return pltpu.async_copy(
                table_hbm.at[recs[par]], rows[par], gsems[par])

        def wait_gather(par):
            pltpu.make_async_copy(
                table_hbm.at[recs[par]], rows[par], gsems[par]).wait()

        def start_out(i, par):
            return pltpu.async_copy(
                outs[par], out_hbm.at[pl.ds(base + i * _B, _B)], osems[par])

        def wait_out(par):
            pltpu.make_async_copy(
                outs[par], out_hbm.at[pl.ds(0, _B)], osems[par]).wait()

        def compute_chunk(i, par):
            rref = rows[par]
            oref = outs[par]
            for g in range(n_groups):
                ids_g = ids_v[pl.ds(i * _B + g * _LANES, _LANES)]
                kshift = (ids_g & 3) << 3

                def qbody(q0, carry, g=g, kshift=kshift):
                    qv = q0 + iot
                    qv = jnp.where(qv >= qdim, qv - qdim, qv)
                    word = plsc.load_gather(rref, [rowvec[g], qv])
                    b = (word >> kshift) & 0xFF
                    colv = qv * chunk
                    coff = b * cstride + colv
                    for c in range(chunk):
                        src = coff if c == 0 else coff + c
                        col = colv if c == 0 else colv + c
                        val = plsc.load_gather(codes_v, [src])
                        plsc.store_scatter(oref, [rowvec[g], col], val)
                    return carry

                lax.fori_loop(0, qdim, qbody, jnp.int32(0))

        start_gather(0, 0)
        start_gather(1, 1)
        ccp.wait()

        for i in range(2):
            wait_gather(i)
            compute_chunk(i, i)
            start_out(i, i)
            start_gather(i + 2, i)

        def pbody(p, carry):
            for par in range(2):
                i = 2 * p + par
                wait_gather(par)
                wait_out(par)
                compute_chunk(i, par)
                start_out(i, par)
                start_gather(i + 2, par)
            return carry

        lax.fori_loop(1, n_chunks // 2 - 1, pbody, jnp.int32(0))

        for k in range(2):
            i = n_chunks - 2 + k
            par = i % 2
            wait_gather(par)
            wait_out(par)
            compute_chunk(i, par)
            start_out(i, par)

        wait_out(0)
        wait_out(1)

    return sc_lookup


def kernel(input, indexes, codes):
    shape = input.shape
    n = input.size
    vocab_rows, qdim = indexes.shape
    cent_rows, qdim2, chunk = codes.shape
    assert qdim == qdim2
    outd = qdim * chunk

    rbytes = (qdim + 127) // 128 * 128

    flat_ids = input.reshape(-1).astype(jnp.int32)
    table = _build_packer(vocab_rows, qdim, rbytes)(indexes)
    codes_flat = jnp.pad(
        codes.reshape(cent_rows, outd),
        ((0, 0), (0, _odd(outd) - outd))).reshape(-1)

    fn = _build_sc_lookup(n, qdim, chunk, rbytes, cent_rows)
    out = fn(flat_ids, table, codes_flat)
    return out.reshape(shape + (outd,))


# speedup vs baseline: 10.2664x; 1.3275x over previous
import functools

import jax
import jax.numpy as jnp
from jax import lax
from jax.experimental import pallas as pl
from jax.experimental.pallas import tpu as pltpu
from jax.experimental.pallas import tpu_sc as plsc

_NUM_CORES = 2
_NUM_SUBCORES = 16
_NUM_WORKERS = _NUM_CORES * _NUM_SUBCORES
_LANES = 16

_B = 32
_PACK_BLK = 2048


def _odd(x):
    return x if x % 2 else x + 1


@functools.lru_cache(maxsize=None)
def _build_packer(vocab_rows, qdim, rbytes):
    rwords = rbytes
    grid = (vocab_rows + _PACK_BLK - 1) // _PACK_BLK
    out_rows = grid * (_PACK_BLK // 4)

    def body(x_ref, o_ref):
        o_ref[...] = pltpu.bitcast(x_ref[...], jnp.int32)

    return pl.pallas_call(
        body,
        grid=(grid,),
        in_specs=[pl.BlockSpec((_PACK_BLK, rbytes), lambda i: (i, 0))],
        out_specs=pl.BlockSpec((_PACK_BLK // 4, rwords), lambda i: (i, 0)),
        out_shape=jax.ShapeDtypeStruct((out_rows, rwords), jnp.int32),
    )


@functools.lru_cache(maxsize=None)
def _build_sc_lookup(n, qdim, chunk, rwords, cent_rows):
    outd = qdim * chunk
    cstride = _odd(outd)
    per_w = n // _NUM_WORKERS
    n_chunks = per_w // _B
    assert per_w * _NUM_WORKERS == n and n_chunks * _B == per_w and n_chunks >= 6

    mesh = plsc.VectorSubcoreMesh(core_axis_name="c", subcore_axis_name="s")

    @functools.partial(
        pl.kernel,
        out_type=jax.ShapeDtypeStruct((n, outd), jnp.float32),
        mesh=mesh,
        scratch_types=[
            pltpu.VMEM((cent_rows * cstride,), jnp.float32),
            pltpu.VMEM((per_w,), jnp.int32),
            pltpu.VMEM((_B,), jnp.int32),
            pltpu.VMEM((_B,), jnp.int32),
            pltpu.VMEM((_B, rwords), jnp.int32),
            pltpu.VMEM((_B, rwords), jnp.int32),
            pltpu.VMEM((_B, outd), jnp.float32),
            pltpu.VMEM((_B, outd), jnp.float32),
            pltpu.SemaphoreType.DMA,
            pltpu.SemaphoreType.DMA,
            pltpu.SemaphoreType.DMA,
            pltpu.SemaphoreType.DMA,
            pltpu.SemaphoreType.DMA,
            pltpu.SemaphoreType.DMA,
        ],
        compiler_params=pltpu.CompilerParams(
            needs_layout_passes=False, use_tc_tiling_on_sc=False),
    )
    def sc_lookup(ids_hbm, table_hbm, codes_hbm, out_hbm,
                  codes_v, ids_v, recs0, recs1, rows0, rows1, out0, out1,
                  sem_c, sem_i, gsem0, gsem1, osem0, osem1):
        wid = lax.axis_index("s") * _NUM_CORES + lax.axis_index("c")
        base = wid * per_w

        recs = (recs0, recs1)
        rows = (rows0, rows1)
        outs = (out0, out1)
        gsems = (gsem0, gsem1)
        osems = (osem0, osem1)

        ccp = pltpu.async_copy(codes_hbm, codes_v, sem_c)
        pltpu.async_copy(ids_hbm.at[pl.ds(base, per_w)], ids_v, sem_i).wait()

        iot = lax.iota(jnp.int32, _LANES)
        n_groups = _B // _LANES
        rowvec = [iot + g * _LANES for g in range(n_groups)]

        def start_gather(i, par):
            for g in range(n_groups):
                ids_g = ids_v[pl.ds(i * _B + g * _LANES, _LANES)]
                recs[par][pl.ds(g * _LANES, _LANES)] = ids_g >> 2
            return pltpu.async_copy(
                table_hbm.at[recs[par]], rows[par], gsems[par])

        def wait_gather(par):
            pltpu.make_async_copy(
                table_hbm.at[recs[par]], rows[par], gsems[par]).wait()

        def start_out(i, par):
            return pltpu.async_copy(
                outs[par], out_hbm.at[pl.ds(base + i * _B, _B)], osems[par])

        def wait_out(par):
            pltpu.make_async_copy(
                outs[par], out_hbm.at[pl.ds(0, _B)], osems[par]).wait()

        def compute_chunk(i, par):
            rref = rows[par]
            oref = outs[par]
            for g in range(n_groups):
                ids_g = ids_v[pl.ds(i * _B + g * _LANES, _LANES)]
                kshift = (ids_g & 3) << 3

                unroll = 5
                assert qdim % unroll == 0

                def qbody(t, carry, g=g, kshift=kshift):
                    for u in range(unroll):
                        qv = t * unroll + u + iot
                        qv = jnp.where(qv >= qdim, qv - qdim, qv)
                        word = plsc.load_gather(rref, [rowvec[g], qv])
                        b = (word >> kshift) & 0xFF
                        colv = qv * chunk
                        coff = b * cstride + colv
                        for c in range(chunk):
                            src = coff if c == 0 else coff + c
                            col = colv if c == 0 else colv + c
                            val = plsc.load_gather(codes_v, [src])
                            plsc.store_scatter(oref, [rowvec[g], col], val)
                    return carry

                lax.fori_loop(0, qdim // unroll, qbody, jnp.int32(0))

        start_gather(0, 0)
        start_gather(1, 1)
        ccp.wait()

        for i in range(2):
            wait_gather(i)
            compute_chunk(i, i)
            start_out(i, i)
            start_gather(i + 2, i)

        def pbody(p, carry):
            for par in range(2):
                i = 2 * p + par
                wait_gather(par)
                wait_out(par)
                compute_chunk(i, par)
                start_out(i, par)
                start_gather(i + 2, par)
            return carry

        lax.fori_loop(1, n_chunks // 2 - 1, pbody, jnp.int32(0))

        for k in range(2):
            i = n_chunks - 2 + k
            par = i % 2
            wait_gather(par)
            wait_out(par)
            compute_chunk(i, par)
            start_out(i, par)

        wait_out(0)
        wait_out(1)

    return sc_lookup


def kernel(input, indexes, codes):
    shape = input.shape
    n = input.size
    vocab_rows, qdim = indexes.shape
    cent_rows, qdim2, chunk = codes.shape
    assert qdim == qdim2
    outd = qdim * chunk

    rbytes = (qdim + 127) // 128 * 128

    flat_ids = input.reshape(-1).astype(jnp.int32)
    table = _build_packer(vocab_rows, qdim, rbytes)(indexes)
    codes_flat = jnp.pad(
        codes.reshape(cent_rows, outd),
        ((0, 0), (0, _odd(outd) - outd))).reshape(-1)

    fn = _build_sc_lookup(n, qdim, chunk, rbytes, cent_rows)
    out = fn(flat_ids, table, codes_flat)
    return out.reshape(shape + (outd,))


# speedup vs baseline: 11.0452x; 1.0759x over previous
import functools

import jax
import jax.numpy as jnp
from jax import lax
from jax.experimental import pallas as pl
from jax.experimental.pallas import tpu as pltpu
from jax.experimental.pallas import tpu_sc as plsc

_NUM_CORES = 2
_NUM_SUBCORES = 16
_NUM_WORKERS = _NUM_CORES * _NUM_SUBCORES
_LANES = 16

_B = 32
_PACK_BLK = 4096


def _odd(x):
    return x if x % 2 else x + 1


@functools.lru_cache(maxsize=None)
def _build_packer(vocab_rows, qdim, rbytes):
    rwords = rbytes
    grid = (vocab_rows + _PACK_BLK - 1) // _PACK_BLK
    out_rows = grid * (_PACK_BLK // 4)

    def body(x_ref, o_ref):
        o_ref[...] = pltpu.bitcast(x_ref[...], jnp.int32)

    return pl.pallas_call(
        body,
        grid=(grid,),
        in_specs=[pl.BlockSpec((_PACK_BLK, rbytes), lambda i: (i, 0))],
        out_specs=pl.BlockSpec((_PACK_BLK // 4, rwords), lambda i: (i, 0)),
        out_shape=jax.ShapeDtypeStruct((out_rows, rwords), jnp.int32),
    )


@functools.lru_cache(maxsize=None)
def _build_sc_lookup(n, qdim, chunk, rwords, cent_rows):
    outd = qdim * chunk
    cstride = _odd(outd)
    per_w = n // _NUM_WORKERS
    n_chunks = per_w // _B
    assert per_w * _NUM_WORKERS == n and n_chunks * _B == per_w and n_chunks >= 6

    mesh = plsc.VectorSubcoreMesh(core_axis_name="c", subcore_axis_name="s")

    @functools.partial(
        pl.kernel,
        out_type=jax.ShapeDtypeStruct((n, outd), jnp.float32),
        mesh=mesh,
        scratch_types=[
            pltpu.VMEM((cent_rows * cstride,), jnp.float32),
            pltpu.VMEM((per_w,), jnp.int32),
            pltpu.VMEM((_B,), jnp.int32),
            pltpu.VMEM((_B,), jnp.int32),
            pltpu.VMEM((_B, rwords), jnp.int32),
            pltpu.VMEM((_B, rwords), jnp.int32),
            pltpu.VMEM((_B, outd), jnp.float32),
            pltpu.VMEM((_B, outd), jnp.float32),
            pltpu.SemaphoreType.DMA,
            pltpu.SemaphoreType.DMA,
            pltpu.SemaphoreType.DMA,
            pltpu.SemaphoreType.DMA,
            pltpu.SemaphoreType.DMA,
            pltpu.SemaphoreType.DMA,
        ],
        compiler_params=pltpu.CompilerParams(
            needs_layout_passes=False, use_tc_tiling_on_sc=False),
    )
    def sc_lookup(ids_hbm, table_hbm, codes_hbm, out_hbm,
                  codes_v, ids_v, recs0, recs1, rows0, rows1, out0, out1,
                  sem_c, sem_i, gsem0, gsem1, osem0, osem1):
        wid = lax.axis_index("s") * _NUM_CORES + lax.axis_index("c")
        base = wid * per_w

        recs = (recs0, recs1)
        rows = (rows0, rows1)
        outs = (out0, out1)
        gsems = (gsem0, gsem1)
        osems = (osem0, osem1)

        ccp = pltpu.async_copy(codes_hbm, codes_v, sem_c)
        pltpu.async_copy(ids_hbm.at[pl.ds(base, per_w)], ids_v, sem_i).wait()

        iot = lax.iota(jnp.int32, _LANES)
        n_groups = _B // _LANES
        rowvec = [iot + g * _LANES for g in range(n_groups)]

        def start_gather(i, par):
            for g in range(n_groups):
                ids_g = ids_v[pl.ds(i * _B + g * _LANES, _LANES)]
                recs[par][pl.ds(g * _LANES, _LANES)] = ids_g >> 2
            return pltpu.async_copy(
                table_hbm.at[recs[par]], rows[par], gsems[par])

        def wait_gather(par):
            pltpu.make_async_copy(
                table_hbm.at[recs[par]], rows[par], gsems[par]).wait()

        def start_out(i, par):
            return pltpu.async_copy(
                outs[par], out_hbm.at[pl.ds(base + i * _B, _B)], osems[par])

        def wait_out(par):
            pltpu.make_async_copy(
                outs[par], out_hbm.at[pl.ds(0, _B)], osems[par]).wait()

        def compute_chunk(i, par):
            rref = rows[par]
            oref = outs[par]
            for g in range(n_groups):
                ids_g = ids_v[pl.ds(i * _B + g * _LANES, _LANES)]
                kshift = (ids_g & 3) << 3

                unroll = 10
                assert qdim % unroll == 0

                def qbody(t, carry, g=g, kshift=kshift):
                    for u in range(unroll):
                        qv = t * unroll + u + iot
                        qv = jnp.where(qv >= qdim, qv - qdim, qv)
                        word = plsc.load_gather(rref, [rowvec[g], qv])
                        b = (word >> kshift) & 0xFF
                        colv = qv * chunk
                        coff = b * cstride + colv
                        for c in range(chunk):
                            src = coff if c == 0 else coff + c
                            col = colv if c == 0 else colv + c
                            val = plsc.load_gather(codes_v, [src])
                            plsc.store_scatter(oref, [rowvec[g], col], val)
                    return carry

                lax.fori_loop(0, qdim // unroll, qbody, jnp.int32(0))

        start_gather(0, 0)
        start_gather(1, 1)
        ccp.wait()

        for i in range(2):
            wait_gather(i)
            compute_chunk(i, i)
            start_out(i, i)
            start_gather(i + 2, i)

        def pbody(p, carry):
            for par in range(2):
                i = 2 * p + par
                wait_gather(par)
                wait_out(par)
                compute_chunk(i, par)
                start_out(i, par)
                start_gather(i + 2, par)
            return carry

        lax.fori_loop(1, n_chunks // 2 - 1, pbody, jnp.int32(0))

        for k in range(2):
            i = n_chunks - 2 + k
            par = i % 2
            wait_gather(par)
            wait_out(par)
            compute_chunk(i, par)
            start_out(i, par)

        wait_out(0)
        wait_out(1)

    return sc_lookup


def kernel(input, indexes, codes):
    shape = input.shape
    n = input.size
    vocab_rows, qdim = indexes.shape
    cent_rows, qdim2, chunk = codes.shape
    assert qdim == qdim2
    outd = qdim * chunk

    rbytes = (qdim + 127) // 128 * 128

    flat_ids = input.reshape(-1).astype(jnp.int32)
    table = _build_packer(vocab_rows, qdim, rbytes)(indexes)
    codes_flat = jnp.pad(
        codes.reshape(cent_rows, outd),
        ((0, 0), (0, _odd(outd) - outd))).reshape(-1)

    fn = _build_sc_lookup(n, qdim, chunk, rbytes, cent_rows)
    out = fn(flat_ids, table, codes_flat)
    return out.reshape(shape + (outd,))
